# baseline (speedup 1.0000x reference)
import functools

import jax
import jax.numpy as jnp
from jax import lax
from jax.experimental import pallas as pl
from jax.experimental.pallas import tpu as pltpu
from jax.experimental.pallas import tpu_sc as plsc

_TBLK = 2304


def _vq_block(x_ref, cbt_ref, idx_ref):
    x = x_ref[:]
    cbt = cbt_ref[:]
    k = cbt.shape[1]

    cnorm = jnp.sum(cbt * cbt, axis=0, keepdims=True)
    dots = jax.lax.dot_general(
        x, cbt, (((1,), (0,)), ((), ())),
        preferred_element_type=jnp.float32,
        precision=jax.lax.Precision.HIGHEST,
    )
    scores = cnorm - 2.0 * dots

    mins = jnp.min(scores, axis=1, keepdims=True)
    iota = jax.lax.broadcasted_iota(jnp.int32, scores.shape, 1)
    loc = jnp.min(jnp.where(scores == mins, iota, k),
                  axis=1, keepdims=True)
    idx_ref[:] = loc.reshape(1, x.shape[0])


@functools.partial(jax.jit, static_argnames=())
def _vq_argmin(flat_x, codebook_t):
    n, d = flat_x.shape
    k = codebook_t.shape[1]
    nblk = n // _TBLK
    idx = pl.pallas_call(
        _vq_block,
        grid=(nblk,),
        in_specs=[
            pl.BlockSpec((_TBLK, d), lambda i: (i, 0)),
            pl.BlockSpec((d, k), lambda i: (0, 0)),
        ],
        out_specs=pl.BlockSpec((1, _TBLK), lambda i: (0, i)),
        out_shape=jax.ShapeDtypeStruct((1, n), jnp.int32),
    )(flat_x, codebook_t)
    return idx


def _make_sc_select(n, d, n_workers, b_per_w):
    mesh = plsc.VectorSubcoreMesh(core_axis_name="c", subcore_axis_name="s")

    @functools.partial(
        pl.kernel, mesh=mesh,
        out_type=[
            jax.ShapeDtypeStruct((n, d), jnp.float32),
            jax.ShapeDtypeStruct((n, d), jnp.float32),
        ],
        scratch_types=[
            pltpu.VMEM((b_per_w,), jnp.int32),
            pltpu.VMEM((b_per_w, 2 * d), jnp.float32),
            pltpu.VMEM((b_per_w, d), jnp.float32),
            pltpu.VMEM((b_per_w, d), jnp.float32),
            pltpu.SemaphoreType.DMA,
        ],
    )
    def sc_select(x_hbm, cb_pad_hbm, idx_hbm, q_hbm, loss_hbm,
                  idx_v, rows_v, x_v, loss_v, sem):
        info = plsc.get_sparse_core_info()
        wid = lax.axis_index("s") * info.num_cores + lax.axis_index("c")
        base = wid * b_per_w
        pltpu.sync_copy(idx_hbm.at[pl.ds(base, b_per_w)], idx_v)
        gather = pltpu.async_copy(cb_pad_hbm.at[idx_v], rows_v, sem)
        pltpu.sync_copy(x_hbm.at[pl.ds(base, b_per_w)], x_v)
        gather.wait()

        nlane = info.num_lanes

        def row_body(r, carry):
            for c in range(d // nlane):
                sl = pl.ds(c * nlane, nlane)
                xv = x_v[r, sl]
                qv = rows_v[r, sl]
                dv = qv - xv
                loss_v[r, sl] = dv * dv
                x_v[r, sl] = xv + dv
            return carry

        lax.fori_loop(0, b_per_w, row_body, 0)

        pltpu.sync_copy(x_v, q_hbm.at[pl.ds(base, b_per_w)])
        pltpu.sync_copy(loss_v, loss_hbm.at[pl.ds(base, b_per_w)])

    return sc_select


def kernel(inputs, codebook):
    b, t, d = inputs.shape
    n = b * t
    half = n // 2
    flat = inputs.reshape(n, d)
    cbt = codebook.T
    cb_pad = jnp.pad(codebook, ((0, 0), (0, d)))
    x0, x1 = flat[:half], flat[half:]
    idx0 = _vq_argmin(x0, cbt)
    sc_select = _make_sc_select(half, d, 32, half // 32)
    q0, l0 = sc_select(x0, cb_pad, idx0.reshape(half))
    idx1 = _vq_argmin(x1, cbt)
    q1, l1 = sc_select(x1, cb_pad, idx1.reshape(half))
    quantized = jnp.concatenate([q0, q1]).reshape(1, b, t, d)
    quantization_loss = jnp.concatenate([l0, l1]).reshape(1, b, t, d)
    nn_idx = jnp.concatenate([idx0, idx1], axis=1).reshape(1, b, t)
    codebook_out = jax.lax.stop_gradient(codebook[None])
    return (quantized, quantization_loss, nn_idx, codebook_out)


# speedup vs baseline: 1.1167x; 1.1167x over previous
import functools

import jax
import jax.numpy as jnp
from jax import lax
from jax.experimental import pallas as pl
from jax.experimental.pallas import tpu as pltpu
from jax.experimental.pallas import tpu_sc as plsc

_TBLK = 2304


def _vq_block(x_ref, cbt_ref, idx_ref):
    x = x_ref[:]
    cbt = cbt_ref[:]
    k = cbt.shape[1]

    cnorm = jnp.sum(cbt * cbt, axis=0, keepdims=True)
    dots = jax.lax.dot_general(
        x, cbt, (((1,), (0,)), ((), ())),
        preferred_element_type=jnp.float32,
        precision=jax.lax.Precision.HIGHEST,
    )
    scores = cnorm - 2.0 * dots

    mins = jnp.min(scores, axis=1, keepdims=True)
    iota = jax.lax.broadcasted_iota(jnp.int32, scores.shape, 1)
    loc = jnp.min(jnp.where(scores == mins, iota, k),
                  axis=1, keepdims=True)
    idx_ref[:] = loc.reshape(1, x.shape[0])


@functools.partial(jax.jit, static_argnames=())
def _vq_argmin(flat_x, codebook_t):
    n, d = flat_x.shape
    k = codebook_t.shape[1]
    nblk = n // _TBLK
    idx = pl.pallas_call(
        _vq_block,
        grid=(nblk,),
        in_specs=[
            pl.BlockSpec((_TBLK, d), lambda i: (i, 0)),
            pl.BlockSpec((d, k), lambda i: (0, 0)),
        ],
        out_specs=pl.BlockSpec((1, _TBLK), lambda i: (0, i)),
        out_shape=jax.ShapeDtypeStruct((1, n), jnp.int32),
    )(flat_x, codebook_t)
    return idx


def _make_sc_select(n, d, n_workers, b_per_w):
    mesh = plsc.VectorSubcoreMesh(core_axis_name="c", subcore_axis_name="s")

    @functools.partial(
        pl.kernel, mesh=mesh,
        out_type=[
            jax.ShapeDtypeStruct((n, d), jnp.float32),
            jax.ShapeDtypeStruct((n, d), jnp.float32),
        ],
        scratch_types=[
            pltpu.VMEM((b_per_w,), jnp.int32),
            pltpu.VMEM((b_per_w, 2 * d), jnp.float32),
            pltpu.VMEM((b_per_w, d), jnp.float32),
            pltpu.VMEM((b_per_w, d), jnp.float32),
            pltpu.SemaphoreType.DMA,
        ],
    )
    def sc_select(x_hbm, cb_pad_hbm, idx_hbm, q_hbm, loss_hbm,
                  idx_v, rows_v, x_v, loss_v, sem):
        info = plsc.get_sparse_core_info()
        wid = lax.axis_index("s") * info.num_cores + lax.axis_index("c")
        base = wid * b_per_w
        pltpu.sync_copy(idx_hbm.at[pl.ds(base, b_per_w)], idx_v)
        gather = pltpu.async_copy(cb_pad_hbm.at[idx_v], rows_v, sem)
        pltpu.sync_copy(x_hbm.at[pl.ds(base, b_per_w)], x_v)
        gather.wait()

        nlane = info.num_lanes

        def row_body(r, carry):
            for c in range(d // nlane):
                sl = pl.ds(c * nlane, nlane)
                qv = rows_v[r, sl]
                dv = qv - x_v[r, sl]
                loss_v[r, sl] = dv * dv
                x_v[r, sl] = qv
            return carry

        lax.fori_loop(0, b_per_w, row_body, 0)

        pltpu.sync_copy(x_v, q_hbm.at[pl.ds(base, b_per_w)])
        pltpu.sync_copy(loss_v, loss_hbm.at[pl.ds(base, b_per_w)])

    return sc_select


def kernel(inputs, codebook):
    b, t, d = inputs.shape
    n = b * t
    flat = inputs.reshape(n, d)
    idx = _vq_argmin(flat, codebook.T)
    idx_flat = idx.reshape(n)
    sc_select = _make_sc_select(n, d, 32, n // 32)
    cb_pad = jnp.pad(codebook, ((0, 0), (0, d)))
    q, loss = sc_select(flat, cb_pad, idx_flat)
    quantized = q.reshape(1, b, t, d)
    quantization_loss = loss.reshape(1, b, t, d)
    nn_idx = idx.reshape(1, b, t)
    codebook_out = jax.lax.stop_gradient(codebook[None])
    return (quantized, quantization_loss, nn_idx, codebook_out)


# speedup vs baseline: 1.1222x; 1.0049x over previous
import functools

import jax
import jax.numpy as jnp
from jax import lax
from jax.experimental import pallas as pl
from jax.experimental.pallas import tpu as pltpu
from jax.experimental.pallas import tpu_sc as plsc

_TBLK = 4608


def _vq_block(x_ref, cbt_ref, idx_ref):
    x = x_ref[:]
    cbt = cbt_ref[:]
    k = cbt.shape[1]

    cnorm = jnp.sum(cbt * cbt, axis=0, keepdims=True)
    dots = jax.lax.dot_general(
        x, cbt, (((1,), (0,)), ((), ())),
        preferred_element_type=jnp.float32,
        precision=jax.lax.Precision.HIGHEST,
    )
    scores = cnorm - 2.0 * dots

    mins = jnp.min(scores, axis=1, keepdims=True)
    iota = jax.lax.broadcasted_iota(jnp.int32, scores.shape, 1)
    loc = jnp.min(jnp.where(scores == mins, iota, k),
                  axis=1, keepdims=True)
    idx_ref[:] = loc.reshape(1, x.shape[0])


@functools.partial(jax.jit, static_argnames=())
def _vq_argmin(flat_x, codebook_t):
    n, d = flat_x.shape
    k = codebook_t.shape[1]
    nblk = n // _TBLK
    idx = pl.pallas_call(
        _vq_block,
        grid=(nblk,),
        in_specs=[
            pl.BlockSpec((_TBLK, d), lambda i: (i, 0)),
            pl.BlockSpec((d, k), lambda i: (0, 0)),
        ],
        out_specs=pl.BlockSpec((1, _TBLK), lambda i: (0, i)),
        out_shape=jax.ShapeDtypeStruct((1, n), jnp.int32),
    )(flat_x, codebook_t)
    return idx


def _make_sc_select(n, d, n_workers, b_per_w):
    mesh = plsc.VectorSubcoreMesh(core_axis_name="c", subcore_axis_name="s")

    @functools.partial(
        pl.kernel, mesh=mesh,
        out_type=[
            jax.ShapeDtypeStruct((n, d), jnp.float32),
            jax.ShapeDtypeStruct((n, d), jnp.float32),
        ],
        scratch_types=[
            pltpu.VMEM((b_per_w,), jnp.int32),
            pltpu.VMEM((b_per_w, 2 * d), jnp.float32),
            pltpu.VMEM((b_per_w, d), jnp.float32),
            pltpu.VMEM((b_per_w, d), jnp.float32),
            pltpu.SemaphoreType.DMA,
        ],
    )
    def sc_select(x_hbm, cb_pad_hbm, idx_hbm, q_hbm, loss_hbm,
                  idx_v, rows_v, x_v, loss_v, sem):
        info = plsc.get_sparse_core_info()
        wid = lax.axis_index("s") * info.num_cores + lax.axis_index("c")
        base = wid * b_per_w
        pltpu.sync_copy(idx_hbm.at[pl.ds(base, b_per_w)], idx_v)
        gather = pltpu.async_copy(cb_pad_hbm.at[idx_v], rows_v, sem)
        pltpu.sync_copy(x_hbm.at[pl.ds(base, b_per_w)], x_v)
        gather.wait()

        nlane = info.num_lanes

        def row_body(r, carry):
            for c in range(d // nlane):
                sl = pl.ds(c * nlane, nlane)
                qv = rows_v[r, sl]
                dv = qv - x_v[r, sl]
                loss_v[r, sl] = dv * dv
                x_v[r, sl] = qv
            return carry

        lax.fori_loop(0, b_per_w, row_body, 0)

        pltpu.sync_copy(x_v, q_hbm.at[pl.ds(base, b_per_w)])
        pltpu.sync_copy(loss_v, loss_hbm.at[pl.ds(base, b_per_w)])

    return sc_select


def kernel(inputs, codebook):
    b, t, d = inputs.shape
    n = b * t
    flat = inputs.reshape(n, d)
    idx = _vq_argmin(flat, codebook.T)
    idx_flat = idx.reshape(n)
    sc_select = _make_sc_select(n, d, 32, n // 32)
    cb_pad = jnp.pad(codebook, ((0, 0), (0, d)))
    q, loss = sc_select(flat, cb_pad, idx_flat)
    quantized = q.reshape(1, b, t, d)
    quantization_loss = loss.reshape(1, b, t, d)
    nn_idx = idx.reshape(1, b, t)
    codebook_out = jax.lax.stop_gradient(codebook[None])
    return (quantized, quantization_loss, nn_idx, codebook_out)
